# baseline (speedup 1.0000x reference)
import functools

import jax
import jax.numpy as jnp
from jax import lax
from jax.experimental import pallas as pl
from jax.experimental.pallas import tpu as pltpu
from jax.experimental.pallas import tpu_sc as plsc

B, N1, N2 = 8, 4096, 1024
C1, C2 = 128, 256
NQ = B * N1
TILE = 1024
EPS = 1e-3

SC_NC, SC_NS, SC_L = 2, 16, 16
SC_NW = SC_NC * SC_NS
SC_QPW = NQ // SC_NW
SC_G = 32
SC_NCHUNK = SC_QPW // SC_G


def _three_nn_body(x1_ref, x2_ref, pk_ref):
    tiles_per_batch = N1 // TILE
    b = pl.program_id(0) // tiles_per_batch
    a = x1_ref[...]
    x2 = x2_ref[0]
    dot = lax.dot_general(a, x2, (((1,), (0,)), ((), ())),
                          preferred_element_type=jnp.float32)
    n1sq = jnp.sum(a * a, axis=1, keepdims=True) * 0.25
    n2sq = jnp.sum(x2 * x2, axis=0, keepdims=True)
    d = n1sq + dot + n2sq
    iota = lax.broadcasted_iota(jnp.int32, d.shape, 1)
    big = jnp.float32(3e38)
    vals, idxs = [], []
    for _ in range(3):
        v = jnp.min(d, axis=1, keepdims=True)
        i = jnp.min(jnp.where(d <= v, iota, N2), axis=1, keepdims=True)
        vals.append(v)
        idxs.append(i)
        d = jnp.where(iota == i, big, d)
    dist = jnp.maximum(jnp.concatenate(vals, axis=1), 1e-10)
    inv = 1.0 / dist
    w = inv / jnp.sum(inv, axis=1, keepdims=True)
    gidx = jnp.concatenate(idxs, axis=1) + b * N2
    wbits = lax.bitcast_convert_type(w, jnp.int32) & -8192
    pk_ref[...] = wbits | gidx


def _three_nn(x1p, x2pt):
    tiles_per_batch = N1 // TILE
    return pl.pallas_call(
        _three_nn_body,
        grid=(NQ // TILE,),
        in_specs=[
            pl.BlockSpec((TILE, 8), lambda i: (i, 0)),
            pl.BlockSpec((1, 8, N2), lambda i: (i // tiles_per_batch, 0, 0)),
        ],
        out_specs=pl.BlockSpec((TILE, 3), lambda i: (i, 0)),
        out_shape=jax.ShapeDtypeStruct((NQ, 3), jnp.int32),
    )(x1p, x2pt)


def _interp_sc(table, packed2d):
    mesh = plsc.VectorSubcoreMesh(core_axis_name="c", subcore_axis_name="s")
    row = 3 * SC_G

    @functools.partial(
        pl.kernel,
        mesh=mesh,
        out_type=jax.ShapeDtypeStruct((NQ, C2 // 2), jnp.int32),
        scratch_types=[
            pltpu.VMEM((SC_NCHUNK, 3 * SC_G), jnp.int32),
            pltpu.VMEM((SC_NCHUNK, 3 * SC_G), jnp.int32),
            pltpu.VMEM((SC_NCHUNK * 3 * SC_G + SC_L,), jnp.float32),
            pltpu.VMEM((2, 3 * SC_G, C2 // 2), jnp.int32),
            pltpu.VMEM((2, SC_G, C2 // 2), jnp.int32),
            pltpu.SemaphoreType.DMA,
            pltpu.SemaphoreType.DMA,
            pltpu.SemaphoreType.DMA,
            pltpu.SemaphoreType.DMA,
        ],
    )
    def k(table_hbm, pk_hbm, out_hbm, pk_v, idx_all, w_all, rows_v, out_v,
          gsem0, gsem1, osem0, osem1):
        gsems = (gsem0, gsem1)
        osems = (osem0, osem1)
        wid = lax.axis_index("s") * SC_NC + lax.axis_index("c")
        qbase0 = wid * SC_QPW

        pltpu.sync_copy(pk_hbm.at[pl.ds(wid * SC_NCHUNK, SC_NCHUNK)], pk_v)

        def unpack(r, carry):
            for j in range(row // SC_L):
                p = pk_v[r, pl.ds(j * SC_L, SC_L)]
                idx_all[r, pl.ds(j * SC_L, SC_L)] = p & 8191
                w_all[pl.ds(r * row + j * SC_L, SC_L)] = (
                    lax.bitcast_convert_type(p & -8192, jnp.float32))
            return carry

        lax.fori_loop(0, SC_NCHUNK, unpack, 0, unroll=2)

        def gather(c, buf):
            return pltpu.async_copy(table_hbm.at[idx_all.at[c]],
                                    rows_v.at[buf], gsems[buf])

        gather(0, 0)

        def pair(p, carry):
            c0 = p * 2
            for bb in range(2):
                c = c0 + bb
                qb = qbase0 + c * SC_G

                @pl.when(c + 1 < SC_NCHUNK)
                def _():
                    gather(c + 1, 1 - bb)

                pltpu.make_async_copy(table_hbm.at[idx_all.at[0]],
                                      rows_v.at[bb], gsems[bb]).wait()

                @pl.when(c >= 2)
                def _():
                    pltpu.make_async_copy(out_v.at[bb],
                                          out_hbm.at[pl.ds(0, SC_G)],
                                          osems[bb]).wait()

                wb = c * 3 * SC_G

                def qloop(g, carry2):
                    wvec = w_all[pl.ds(wb + 3 * g, SC_L)]
                    w0 = wvec[0]
                    w1 = wvec[1]
                    w2 = wvec[2]
                    himask = jnp.int32(-65536)

                    def bflo(p):
                        return lax.bitcast_convert_type(
                            lax.shift_left(p, 16), jnp.float32)

                    def bfhi(p):
                        return lax.bitcast_convert_type(p & himask,
                                                        jnp.float32)

                    for j in range(C2 // 2 // SC_L):
                        sl = pl.ds(SC_L * j, SC_L)
                        p0 = rows_v[bb, 3 * g, sl]
                        p1 = rows_v[bb, 3 * g + 1, sl]
                        p2 = rows_v[bb, 3 * g + 2, sl]
                        olo = w0 * bflo(p0) + w1 * bflo(p1) + w2 * bflo(p2)
                        ohi = w0 * bfhi(p0) + w1 * bfhi(p1) + w2 * bfhi(p2)
                        out_v[bb, g, sl] = (
                            lax.shift_right_logical(
                                lax.bitcast_convert_type(olo, jnp.int32), 16)
                            | (lax.bitcast_convert_type(ohi, jnp.int32)
                               & himask))
                    return carry2

                lax.fori_loop(0, SC_G, qloop, 0, unroll=4)
                pltpu.async_copy(out_v.at[bb], out_hbm.at[pl.ds(qb, SC_G)],
                                 osems[bb])
            return carry

        lax.fori_loop(0, SC_NCHUNK // 2, pair, 0)
        for bb in range(2):
            pltpu.make_async_copy(out_v.at[bb], out_hbm.at[pl.ds(0, SC_G)],
                                  osems[bb]).wait()

    return k(table, packed2d)


def _mlp1_body(interp_ref, p1_ref, w0a_ref, w0b_ref, b0_ref,
               y_ref, stats_ref, acc_ref):
    i = pl.program_id(0)
    y = (lax.dot_general(interp_ref[...], w0a_ref[...],
                         (((1,), (0,)), ((), ())),
                         preferred_element_type=jnp.float32)
         + lax.dot_general(p1_ref[0], w0b_ref[...], (((1,), (0,)), ((), ())),
                           preferred_element_type=jnp.float32)
         + b0_ref[...])
    y_ref[...] = y.astype(jnp.bfloat16)
    st = jnp.concatenate([jnp.sum(y, axis=0, keepdims=True),
                          jnp.sum(y * y, axis=0, keepdims=True)], axis=0)

    @pl.when(i == 0)
    def _():
        acc_ref[...] = st

    @pl.when(i > 0)
    def _():
        acc_ref[...] += st

    @pl.when(i == pl.num_programs(0) - 1)
    def _():
        stats_ref[...] = acc_ref[...]


def _mlp1(interp, points1, w0a, w0b, b0):
    tiles_per_batch = N1 // TILE
    return pl.pallas_call(
        _mlp1_body,
        grid=(NQ // TILE,),
        in_specs=[
            pl.BlockSpec((TILE, C2), lambda i: (i, 0)),
            pl.BlockSpec((1, TILE, C1), lambda i: (i // tiles_per_batch,
                                                   i % tiles_per_batch, 0)),
            pl.BlockSpec((C2, C2), lambda i: (0, 0)),
            pl.BlockSpec((C1, C2), lambda i: (0, 0)),
            pl.BlockSpec((1, C2), lambda i: (0, 0)),
        ],
        out_specs=[
            pl.BlockSpec((TILE, C2), lambda i: (i, 0)),
            pl.BlockSpec((2, C2), lambda i: (0, 0)),
        ],
        out_shape=[
            jax.ShapeDtypeStruct((NQ, C2), jnp.bfloat16),
            jax.ShapeDtypeStruct((2, C2), jnp.float32),
        ],
        scratch_shapes=[pltpu.VMEM((2, C2), jnp.float32)],
    )(interp, points1, w0a, w0b, b0)


def _mlp2_body(y_ref, st_ref, g0_ref, be0_ref, w1_ref, b1_ref,
               h_ref, stats_ref, acc_ref):
    i = pl.program_id(0)
    st = st_ref[...]
    mean = st[0:1] * (1.0 / NQ)
    var = st[1:2] * (1.0 / NQ) - mean * mean
    scale = g0_ref[...] * lax.rsqrt(var + EPS)
    shift = be0_ref[...] - mean * scale
    z = jnp.maximum(y_ref[...].astype(jnp.float32) * scale + shift, 0.0)
    h = lax.dot_general(z, w1_ref[...], (((1,), (0,)), ((), ())),
                        preferred_element_type=jnp.float32) + b1_ref[...]
    h_ref[...] = h
    st2 = jnp.concatenate([jnp.sum(h, axis=0, keepdims=True),
                           jnp.sum(h * h, axis=0, keepdims=True)], axis=0)

    @pl.when(i == 0)
    def _():
        acc_ref[...] = st2

    @pl.when(i > 0)
    def _():
        acc_ref[...] += st2

    @pl.when(i == pl.num_programs(0) - 1)
    def _():
        stats_ref[...] = acc_ref[...]


def _mlp2(y, st0, g0, be0, w1, b1):
    return pl.pallas_call(
        _mlp2_body,
        grid=(NQ // TILE,),
        in_specs=[
            pl.BlockSpec((TILE, C2), lambda i: (i, 0)),
            pl.BlockSpec((2, C2), lambda i: (0, 0)),
            pl.BlockSpec((1, C2), lambda i: (0, 0)),
            pl.BlockSpec((1, C2), lambda i: (0, 0)),
            pl.BlockSpec((C2, C1), lambda i: (0, 0)),
            pl.BlockSpec((1, C1), lambda i: (0, 0)),
        ],
        out_specs=[
            pl.BlockSpec((TILE, C1), lambda i: (i, 0)),
            pl.BlockSpec((2, C1), lambda i: (0, 0)),
        ],
        out_shape=[
            jax.ShapeDtypeStruct((NQ, C1), jnp.float32),
            jax.ShapeDtypeStruct((2, C1), jnp.float32),
        ],
        scratch_shapes=[pltpu.VMEM((2, C1), jnp.float32)],
    )(y, st0, g0, be0, w1, b1)


TILE3 = 2048


def _mlp3_body(h_ref, st_ref, g1_ref, be1_ref, out_ref):
    st = st_ref[...]
    mean = st[0:1] * (1.0 / NQ)
    var = st[1:2] * (1.0 / NQ) - mean * mean
    scale = g1_ref[...] * lax.rsqrt(var + EPS)
    shift = be1_ref[...] - mean * scale
    out_ref[...] = jnp.maximum(h_ref[...] * scale + shift, 0.0)


def _mlp3(h, st1, g1, be1):
    return pl.pallas_call(
        _mlp3_body,
        grid=(NQ // TILE3,),
        in_specs=[
            pl.BlockSpec((TILE3, C1), lambda i: (i, 0)),
            pl.BlockSpec((2, C1), lambda i: (0, 0)),
            pl.BlockSpec((1, C1), lambda i: (0, 0)),
            pl.BlockSpec((1, C1), lambda i: (0, 0)),
        ],
        out_specs=pl.BlockSpec((TILE3, C1), lambda i: (i, 0)),
        out_shape=jax.ShapeDtypeStruct((NQ, C1), jnp.float32),
    )(h, st1, g1, be1)


def kernel(xyz1, xyz2, points1, points2, W0, b0, gamma0, beta0,
           W1, b1, gamma1, beta1):
    x1p = jnp.pad(-2.0 * xyz1, ((0, 0), (0, 0), (0, 5))).reshape(NQ, 8)
    x2pt = jnp.pad(xyz2, ((0, 0), (0, 0), (0, 5))).transpose(0, 2, 1)
    pk = _three_nn(x1p, x2pt)
    table_i = lax.bitcast_convert_type(
        points2.astype(jnp.bfloat16).reshape(B * N2, C2 // 2, 2), jnp.int32)
    interp_i = _interp_sc(table_i,
                          pk.reshape(NQ * 3 // (3 * SC_G), 3 * SC_G))
    interp = lax.bitcast_convert_type(
        interp_i, jnp.bfloat16).reshape(NQ, C2)
    y, st0 = _mlp1(interp, points1, W0[:C2].astype(jnp.bfloat16), W0[C2:],
                   b0.reshape(1, C2))
    h, st1 = _mlp2(y, st0, gamma0.reshape(1, C2), beta0.reshape(1, C2),
                   W1, b1.reshape(1, C1))
    out = _mlp3(h, st1, gamma1.reshape(1, C1), beta1.reshape(1, C1))
    return out.reshape(B, N1, C1)


# speedup vs baseline: 1.3075x; 1.3075x over previous
import functools

import jax
import jax.numpy as jnp
from jax import lax
from jax.experimental import pallas as pl
from jax.experimental.pallas import tpu as pltpu
from jax.experimental.pallas import tpu_sc as plsc

B, N1, N2 = 8, 4096, 1024
C1, C2 = 128, 256
NQ = B * N1
TILE = 1024
EPS = 1e-3

SC_NC, SC_NS, SC_L = 2, 16, 16
SC_NW = SC_NC * SC_NS
SC_QPW = NQ // SC_NW
SC_G = 32
SC_NCHUNK = SC_QPW // SC_G


def _three_nn_body(x1_ref, x2_ref, pk_ref):
    tiles_per_batch = N1 // TILE
    b = pl.program_id(0) // tiles_per_batch
    a = x1_ref[...]
    x2 = x2_ref[0]
    dot = lax.dot_general(a, x2, (((1,), (0,)), ((), ())),
                          preferred_element_type=jnp.float32)
    n1sq = jnp.sum(a * a, axis=1, keepdims=True) * 0.25
    n2sq = jnp.sum(x2 * x2, axis=0, keepdims=True)
    d = n1sq + dot + n2sq
    iota = lax.broadcasted_iota(jnp.int32, d.shape, 1)
    big = jnp.float32(3e38)
    vals, idxs = [], []
    for _ in range(3):
        v = jnp.min(d, axis=1, keepdims=True)
        i = jnp.min(jnp.where(d <= v, iota, N2), axis=1, keepdims=True)
        vals.append(v)
        idxs.append(i)
        d = jnp.where(iota == i, big, d)
    dist = jnp.maximum(jnp.concatenate(vals, axis=1), 1e-10)
    inv = 1.0 / dist
    w = inv / jnp.sum(inv, axis=1, keepdims=True)
    gidx = jnp.concatenate(idxs, axis=1) + b * N2
    wbits = lax.bitcast_convert_type(w, jnp.int32) & -8192
    pk_ref[...] = wbits | gidx


def _three_nn(x1p, x2pt):
    tiles_per_batch = N1 // TILE
    return pl.pallas_call(
        _three_nn_body,
        grid=(NQ // TILE,),
        in_specs=[
            pl.BlockSpec((TILE, 8), lambda i: (i, 0)),
            pl.BlockSpec((1, 8, N2), lambda i: (i // tiles_per_batch, 0, 0)),
        ],
        out_specs=pl.BlockSpec((TILE, 3), lambda i: (i, 0)),
        out_shape=jax.ShapeDtypeStruct((NQ, 3), jnp.int32),
    )(x1p, x2pt)


def _interp_sc(table, packed2d):
    mesh = plsc.VectorSubcoreMesh(core_axis_name="c", subcore_axis_name="s")
    row = 3 * SC_G

    @functools.partial(
        pl.kernel,
        mesh=mesh,
        out_type=jax.ShapeDtypeStruct((NQ, C2), jnp.float32),
        scratch_types=[
            pltpu.VMEM((SC_NCHUNK, 3 * SC_G), jnp.int32),
            pltpu.VMEM((SC_NCHUNK, 3 * SC_G), jnp.int32),
            pltpu.VMEM((SC_NCHUNK * 3 * SC_G + SC_L,), jnp.float32),
            pltpu.VMEM((2, 3 * SC_G, C2), jnp.float32),
            pltpu.VMEM((2, SC_G, C2), jnp.float32),
            pltpu.SemaphoreType.DMA,
            pltpu.SemaphoreType.DMA,
            pltpu.SemaphoreType.DMA,
            pltpu.SemaphoreType.DMA,
        ],
    )
    def k(table_hbm, pk_hbm, out_hbm, pk_v, idx_all, w_all, rows_v, out_v,
          gsem0, gsem1, osem0, osem1):
        gsems = (gsem0, gsem1)
        osems = (osem0, osem1)
        wid = lax.axis_index("s") * SC_NC + lax.axis_index("c")
        qbase0 = wid * SC_QPW

        pltpu.sync_copy(pk_hbm.at[pl.ds(wid * SC_NCHUNK, SC_NCHUNK)], pk_v)

        def unpack(r, carry):
            for j in range(row // SC_L):
                p = pk_v[r, pl.ds(j * SC_L, SC_L)]
                idx_all[r, pl.ds(j * SC_L, SC_L)] = p & 8191
                w_all[pl.ds(r * row + j * SC_L, SC_L)] = (
                    lax.bitcast_convert_type(p & -8192, jnp.float32))
            return carry

        lax.fori_loop(0, SC_NCHUNK, unpack, 0, unroll=2)

        def gather(c, buf):
            return pltpu.async_copy(table_hbm.at[idx_all.at[c]],
                                    rows_v.at[buf], gsems[buf])

        gather(0, 0)

        def pair(p, carry):
            c0 = p * 2
            for bb in range(2):
                c = c0 + bb
                qb = qbase0 + c * SC_G

                @pl.when(c + 1 < SC_NCHUNK)
                def _():
                    gather(c + 1, 1 - bb)

                pltpu.make_async_copy(table_hbm.at[idx_all.at[0]],
                                      rows_v.at[bb], gsems[bb]).wait()

                @pl.when(c >= 2)
                def _():
                    pltpu.make_async_copy(out_v.at[bb],
                                          out_hbm.at[pl.ds(0, SC_G)],
                                          osems[bb]).wait()

                wb = c * 3 * SC_G

                def qloop(g, carry2):
                    wvec = w_all[pl.ds(wb + 3 * g, SC_L)]
                    w0 = wvec[0]
                    w1 = wvec[1]
                    w2 = wvec[2]
                    for j in range(C2 // SC_L):
                        sl = pl.ds(SC_L * j, SC_L)
                        out_v[bb, g, sl] = (w0 * rows_v[bb, 3 * g, sl]
                                            + w1 * rows_v[bb, 3 * g + 1, sl]
                                            + w2 * rows_v[bb, 3 * g + 2, sl])
                    return carry2

                lax.fori_loop(0, SC_G, qloop, 0, unroll=4)
                pltpu.async_copy(out_v.at[bb], out_hbm.at[pl.ds(qb, SC_G)],
                                 osems[bb])
            return carry

        lax.fori_loop(0, SC_NCHUNK // 2, pair, 0)
        for bb in range(2):
            pltpu.make_async_copy(out_v.at[bb], out_hbm.at[pl.ds(0, SC_G)],
                                  osems[bb]).wait()

    return k(table, packed2d)


def _mlp1_body(interp_ref, p1_ref, w0a_ref, w0b_ref, b0_ref,
               y_ref, stats_ref, acc_ref):
    i = pl.program_id(0)
    y = (lax.dot_general(interp_ref[...], w0a_ref[...],
                         (((1,), (0,)), ((), ())),
                         preferred_element_type=jnp.float32)
         + lax.dot_general(p1_ref[0], w0b_ref[...], (((1,), (0,)), ((), ())),
                           preferred_element_type=jnp.float32)
         + b0_ref[...])
    y_ref[...] = y.astype(jnp.bfloat16)
    st = jnp.concatenate([jnp.sum(y, axis=0, keepdims=True),
                          jnp.sum(y * y, axis=0, keepdims=True)], axis=0)

    @pl.when(i == 0)
    def _():
        acc_ref[...] = st

    @pl.when(i > 0)
    def _():
        acc_ref[...] += st

    @pl.when(i == pl.num_programs(0) - 1)
    def _():
        stats_ref[...] = acc_ref[...]


def _mlp1(interp, points1, w0a, w0b, b0):
    tiles_per_batch = N1 // TILE
    return pl.pallas_call(
        _mlp1_body,
        grid=(NQ // TILE,),
        in_specs=[
            pl.BlockSpec((TILE, C2), lambda i: (i, 0)),
            pl.BlockSpec((1, TILE, C1), lambda i: (i // tiles_per_batch,
                                                   i % tiles_per_batch, 0)),
            pl.BlockSpec((C2, C2), lambda i: (0, 0)),
            pl.BlockSpec((C1, C2), lambda i: (0, 0)),
            pl.BlockSpec((1, C2), lambda i: (0, 0)),
        ],
        out_specs=[
            pl.BlockSpec((TILE, C2), lambda i: (i, 0)),
            pl.BlockSpec((2, C2), lambda i: (0, 0)),
        ],
        out_shape=[
            jax.ShapeDtypeStruct((NQ, C2), jnp.bfloat16),
            jax.ShapeDtypeStruct((2, C2), jnp.float32),
        ],
        scratch_shapes=[pltpu.VMEM((2, C2), jnp.float32)],
    )(interp, points1, w0a, w0b, b0)


def _mlp2_body(y_ref, st_ref, g0_ref, be0_ref, w1_ref, b1_ref,
               h_ref, stats_ref, acc_ref):
    i = pl.program_id(0)
    st = st_ref[...]
    mean = st[0:1] * (1.0 / NQ)
    var = st[1:2] * (1.0 / NQ) - mean * mean
    scale = g0_ref[...] * lax.rsqrt(var + EPS)
    shift = be0_ref[...] - mean * scale
    z = jnp.maximum(y_ref[...].astype(jnp.float32) * scale + shift, 0.0)
    h = lax.dot_general(z, w1_ref[...], (((1,), (0,)), ((), ())),
                        preferred_element_type=jnp.float32) + b1_ref[...]
    h_ref[...] = h
    st2 = jnp.concatenate([jnp.sum(h, axis=0, keepdims=True),
                           jnp.sum(h * h, axis=0, keepdims=True)], axis=0)

    @pl.when(i == 0)
    def _():
        acc_ref[...] = st2

    @pl.when(i > 0)
    def _():
        acc_ref[...] += st2

    @pl.when(i == pl.num_programs(0) - 1)
    def _():
        stats_ref[...] = acc_ref[...]


def _mlp2(y, st0, g0, be0, w1, b1):
    return pl.pallas_call(
        _mlp2_body,
        grid=(NQ // TILE,),
        in_specs=[
            pl.BlockSpec((TILE, C2), lambda i: (i, 0)),
            pl.BlockSpec((2, C2), lambda i: (0, 0)),
            pl.BlockSpec((1, C2), lambda i: (0, 0)),
            pl.BlockSpec((1, C2), lambda i: (0, 0)),
            pl.BlockSpec((C2, C1), lambda i: (0, 0)),
            pl.BlockSpec((1, C1), lambda i: (0, 0)),
        ],
        out_specs=[
            pl.BlockSpec((TILE, C1), lambda i: (i, 0)),
            pl.BlockSpec((2, C1), lambda i: (0, 0)),
        ],
        out_shape=[
            jax.ShapeDtypeStruct((NQ, C1), jnp.float32),
            jax.ShapeDtypeStruct((2, C1), jnp.float32),
        ],
        scratch_shapes=[pltpu.VMEM((2, C1), jnp.float32)],
    )(y, st0, g0, be0, w1, b1)


TILE3 = 2048


def _mlp3_body(h_ref, st_ref, g1_ref, be1_ref, out_ref):
    st = st_ref[...]
    mean = st[0:1] * (1.0 / NQ)
    var = st[1:2] * (1.0 / NQ) - mean * mean
    scale = g1_ref[...] * lax.rsqrt(var + EPS)
    shift = be1_ref[...] - mean * scale
    out_ref[...] = jnp.maximum(h_ref[...] * scale + shift, 0.0)


def _mlp3(h, st1, g1, be1):
    return pl.pallas_call(
        _mlp3_body,
        grid=(NQ // TILE3,),
        in_specs=[
            pl.BlockSpec((TILE3, C1), lambda i: (i, 0)),
            pl.BlockSpec((2, C1), lambda i: (0, 0)),
            pl.BlockSpec((1, C1), lambda i: (0, 0)),
            pl.BlockSpec((1, C1), lambda i: (0, 0)),
        ],
        out_specs=pl.BlockSpec((TILE3, C1), lambda i: (i, 0)),
        out_shape=jax.ShapeDtypeStruct((NQ, C1), jnp.float32),
    )(h, st1, g1, be1)


def kernel(xyz1, xyz2, points1, points2, W0, b0, gamma0, beta0,
           W1, b1, gamma1, beta1):
    x1p = jnp.pad(-2.0 * xyz1, ((0, 0), (0, 0), (0, 5))).reshape(NQ, 8)
    x2pt = jnp.pad(xyz2, ((0, 0), (0, 0), (0, 5))).transpose(0, 2, 1)
    pk = _three_nn(x1p, x2pt)
    interp = _interp_sc(points2.reshape(B * N2, C2),
                        pk.reshape(NQ * 3 // (3 * SC_G), 3 * SC_G))
    y, st0 = _mlp1(interp, points1, W0[:C2], W0[C2:], b0.reshape(1, C2))
    h, st1 = _mlp2(y, st0, gamma0.reshape(1, C2), beta0.reshape(1, C2),
                   W1, b1.reshape(1, C1))
    out = _mlp3(h, st1, gamma1.reshape(1, C1), beta1.reshape(1, C1))
    return out.reshape(B, N1, C1)
